# baseline (speedup 1.0000x reference)
import functools

import jax
import jax.numpy as jnp
from jax import lax
from jax.experimental import pallas as pl
from jax.experimental.pallas import tpu as pltpu
from jax.experimental.pallas import tpu_sc as plsc

B = 16384
F = 26
V = 100000
D = 32
N = B * F

NC = 2
NS = 16
NW = NC * NS
N_PER_W = N // NW
CHUNK = 1024
NCHUNK = N_PER_W // CHUNK

_mesh = plsc.VectorSubcoreMesh(core_axis_name="c", subcore_axis_name="s")


@functools.partial(
    pl.kernel,
    out_type=jax.ShapeDtypeStruct((N, D), jnp.float32),
    mesh=_mesh,
    scratch_types=[
        pltpu.VMEM((CHUNK,), jnp.int32),
        pltpu.VMEM((CHUNK, D), jnp.float32),
        pltpu.SemaphoreType.DMA,
    ],
    compiler_params=pltpu.CompilerParams(use_tc_tiling_on_sc=False),
)
def _gather_rows(table_hbm, idx_hbm, out_hbm, idx_v, rows_v, sem):
    wid = lax.axis_index("s") * NC + lax.axis_index("c")
    base = wid * N_PER_W

    def body(i, carry):
        off = base + i * CHUNK
        pltpu.sync_copy(idx_hbm.at[pl.ds(off, CHUNK)], idx_v)
        pltpu.async_copy(table_hbm.at[idx_v], rows_v, sem).wait()
        pltpu.sync_copy(rows_v, out_hbm.at[pl.ds(off, CHUNK)])
        return carry

    lax.fori_loop(0, NCHUNK, body, 0)


def kernel(x, W):
    idx = (x.astype(jnp.int32)
           + (jnp.arange(F, dtype=jnp.int32) * (V + 1))[None, :]).reshape(N)
    table = W.reshape(F * (V + 1), D)
    out = _gather_rows(table, idx)
    return out.reshape(B, F, D)


# speedup vs baseline: 26.5349x; 26.5349x over previous
import functools

import jax
import jax.numpy as jnp
from jax import lax
from jax.experimental import pallas as pl
from jax.experimental.pallas import tpu as pltpu
from jax.experimental.pallas import tpu_sc as plsc

B = 16384
F = 26
V = 100000
D = 32

NC = 2
NS = 16
NW = NC * NS
PLANES = F * D
PER_W = PLANES // NW
Q = 4
BQ = B // Q
L = 16

_mesh = plsc.VectorSubcoreMesh(core_axis_name="c", subcore_axis_name="s")


@functools.partial(
    pl.kernel,
    out_type=jax.ShapeDtypeStruct((F, D, B), jnp.float32),
    mesh=_mesh,
    scratch_types=[
        pltpu.VMEM((V + 1,), jnp.float32),
        pltpu.VMEM((BQ,), jnp.int32),
        pltpu.VMEM((BQ,), jnp.float32),
    ],
    compiler_params=pltpu.CompilerParams(needs_layout_passes=False),
)
def _plane_gather(x_t_hbm, w_t_hbm, out_hbm, plane_v, idx_v, outq_v):
    wid = lax.axis_index("s") * NC + lax.axis_index("c")

    def plane_body(p, carry):
        pidx = wid * PER_W + p
        f = pidx // D
        d = pidx % D
        pltpu.sync_copy(w_t_hbm.at[f, d], plane_v)

        def q_body(q, qcarry):
            pltpu.sync_copy(x_t_hbm.at[f, pl.ds(q * BQ, BQ)], idx_v)

            def g_body(i, gcarry):
                ii = idx_v[pl.ds(i * L, L)]
                outq_v[pl.ds(i * L, L)] = plsc.load_gather(plane_v, [ii])
                return gcarry

            lax.fori_loop(0, BQ // L, g_body, 0, unroll=8)
            pltpu.sync_copy(outq_v, out_hbm.at[f, d, pl.ds(q * BQ, BQ)])
            return qcarry

        lax.fori_loop(0, Q, q_body, 0)
        return carry

    lax.fori_loop(0, PER_W, plane_body, 0)


def kernel(x, W):
    out_t = _plane_gather(x.T.astype(jnp.int32), W.transpose(0, 2, 1))
    return out_t.transpose(2, 0, 1)
